# Initial kernel scaffold; baseline (speedup 1.0000x reference)
import jax
import jax.numpy as jnp
from jax.experimental import pallas as pl


def kernel(x_user, edge_weight, params, edge_index, lengths):
    raise NotImplementedError("write your pallas kernel here")


# speedup vs baseline: 1.4653x; 1.4653x over previous
import functools
import math

import jax
import jax.numpy as jnp
from jax import lax
from jax.experimental import pallas as pl
from jax.experimental.pallas import tpu as pltpu
from jax.experimental.pallas import tpu_sc as plsc

B = 16; T = 512; P = 4; N = B * T * P; E = 262144
D = 256; H = 8; DH = D // H; FF = 1024; PROJ = 32
BT = B * T

_DEV_SEG_JNP = False
_INTERPRET = False


def _mm_body(x_ref, w_ref, o_ref):
    o_ref[...] = jnp.dot(x_ref[...], w_ref[...],
                         preferred_element_type=jnp.float32)


def _matmul(x, w, block_rows=1024):
    n, k = x.shape
    _, m = w.shape
    return pl.pallas_call(
        _mm_body,
        grid=(n // block_rows,),
        in_specs=[pl.BlockSpec((block_rows, k), lambda i: (i, 0)),
                  pl.BlockSpec((k, m), lambda i: (0, 0))],
        out_specs=pl.BlockSpec((block_rows, m), lambda i: (i, 0)),
        out_shape=jax.ShapeDtypeStruct((n, m), jnp.float32),
        interpret=_INTERPRET,
    )(x, w)


def _h_mm_body(p0_ref, p1_ref, s1_ref, b1_ref, w_ref, o_ref):
    h = jnp.maximum(p0_ref[...] + p1_ref[...] + s1_ref[...] + b1_ref[...], 0.0)
    o_ref[...] = jnp.dot(h, w_ref[...], preferred_element_type=jnp.float32)


def _h_matmul(p0, p1, s1, b1, w, block_rows=1024):
    n, k = p0.shape
    _, m = w.shape
    bs = lambda: pl.BlockSpec((block_rows, k), lambda i: (i, 0))
    return pl.pallas_call(
        _h_mm_body,
        grid=(n // block_rows,),
        in_specs=[bs(), bs(), bs(),
                  pl.BlockSpec((1, k), lambda i: (0, 0)),
                  pl.BlockSpec((k, m), lambda i: (0, 0))],
        out_specs=pl.BlockSpec((block_rows, m), lambda i: (i, 0)),
        out_shape=jax.ShapeDtypeStruct((n, m), jnp.float32),
        interpret=_INTERPRET,
    )(p0, p1, s1, b1, w)


def _lnk(x, g, b, eps=1e-12):
    m = jnp.mean(x, axis=-1, keepdims=True)
    v = jnp.mean((x - m) ** 2, axis=-1, keepdims=True)
    return (x - m) / jnp.sqrt(v + eps) * g + b


def _dot(a, b_):
    return jnp.dot(a, b_, preferred_element_type=jnp.float32)


def _enc_body(len_ref, f0, f1, s2, b2, padv, pos, lng, lnb,
              wq, bq, wk, bk, wv, bv, wo, bo, l1g, l1b,
              wf1, bf1, wf2, bf2, l2g, l2b, out):
    bidx = pl.program_id(0)
    L = len_ref[bidx]
    f = f0[0] + f1[0] + s2[0] + b2[...]
    tidx = lax.broadcasted_iota(jnp.int32, (T, 1), 0)
    mask = tidx < L
    x = jnp.where(mask, f, padv[...])
    x = _lnk(x + pos[...], lng[...], lnb[...])
    xln = _lnk(x, lng[...], lnb[...])
    scores = jnp.where(mask, 0.0, -10000.0).reshape(1, T)
    mf = mask.astype(jnp.float32)
    sm = jnp.maximum(jnp.sum(mf), 1e-12)
    inv = 1.0 / math.sqrt(DH)

    for a, xa in ((0, x), (1, xln)):
        q = _dot(xa, wq[...]) + bq[...]
        k = _dot(xa, wk[...]) + bk[...]
        v = _dot(xa, wv[...]) + bv[...]
        heads = []
        for h in range(H):
            qh = q[:, h * DH:(h + 1) * DH]
            kh = k[:, h * DH:(h + 1) * DH]
            vh = v[:, h * DH:(h + 1) * DH]
            att = lax.dot_general(qh, kh, (((1,), (1,)), ((), ())),
                                  preferred_element_type=jnp.float32) * inv
            att = att + scores
            att = att - jnp.max(att, axis=-1, keepdims=True)
            ew = jnp.exp(att)
            ew = ew / jnp.sum(ew, axis=-1, keepdims=True)
            heads.append(_dot(ew, vh))
        o = jnp.concatenate(heads, axis=1)
        attn = _dot(o, wo[...]) + bo[...]
        x1 = _lnk(xa + attn, l1g[...], l1b[...])
        ffh = jax.nn.gelu(_dot(x1, wf1[...]) + bf1[...])
        ff = _dot(ffh, wf2[...]) + bf2[...]
        x2 = _lnk(x1 + ff, l2g[...], l2b[...])
        out[0, a, :] = jnp.sum(x2 * mf, axis=0) / sm


def _encoder(lengths, f0, f1, s2p0, b2, padv, pos, p):
    row = lambda: pl.BlockSpec((1, T, D), lambda b: (b, 0, 0))
    cst = lambda shp: pl.BlockSpec(shp, lambda b: tuple(0 for _ in shp))
    w_names = ['Wq', 'bq', 'Wk', 'bk', 'Wv', 'bv', 'Wo', 'bo',
               'ln1_g', 'ln1_b', 'Wf1', 'bf1', 'Wf2', 'bf2', 'ln2_g', 'ln2_b']
    wvals = []
    wspecs = []
    for nm in w_names:
        arr = p[nm]
        if arr.ndim == 1:
            arr = arr.reshape(1, -1)
        wvals.append(arr)
        wspecs.append(cst(arr.shape))
    lng = p['ln_g'].reshape(1, D)
    lnb = p['ln_b'].reshape(1, D)
    return pl.pallas_call(
        _enc_body,
        grid=(B,),
        in_specs=[pl.BlockSpec(memory_space=pltpu.SMEM),
                  row(), row(), row(),
                  cst((1, D)), cst((1, D)), cst((T, D)),
                  cst((1, D)), cst((1, D))] + wspecs,
        out_specs=pl.BlockSpec((1, 2, D), lambda b: (b, 0, 0)),
        out_shape=jax.ShapeDtypeStruct((B, 2, D), jnp.float32),
        interpret=_INTERPRET,
    )(lengths, f0, f1, s2p0, b2, padv, pos, lng, lnb, *wvals)


def _head_body(lg_ref, wp1, bp1, wp2, bp2, out_ref):
    l = lg_ref[...]
    r1 = jnp.maximum(_dot(l, wp1[...]) + bp1[...], 0.0)
    pr = _dot(r1, wp2[...]) + bp2[...]
    nrm = jnp.sqrt(jnp.sum(pr * pr, axis=-1, keepdims=True))
    out_ref[...] = pr / jnp.maximum(nrm, 1e-12)


def _head(logits, p):
    return pl.pallas_call(
        _head_body,
        in_specs=[pl.BlockSpec((2 * B, D), lambda: (0, 0)),
                  pl.BlockSpec((D, D), lambda: (0, 0)),
                  pl.BlockSpec((1, D), lambda: (0, 0)),
                  pl.BlockSpec((D, PROJ), lambda: (0, 0)),
                  pl.BlockSpec((1, PROJ), lambda: (0, 0))],
        out_specs=pl.BlockSpec((2 * B, PROJ), lambda: (0, 0)),
        out_shape=jax.ShapeDtypeStruct((2 * B, PROJ), jnp.float32),
        interpret=_INTERPRET,
    )(logits, p['Wp1'].reshape(D, D), p['bp1'].reshape(1, D),
      p['Wp2'].reshape(D, PROJ), p['bp2'].reshape(1, PROJ))


EW = E // 32
CH = 2048
CHS = 11
EB = 2048
GB = 32


def _seg_sc(m, src, dst, w, nout, p0):
    nchunks = nout // CH
    mesh = plsc.VectorSubcoreMesh(core_axis_name="c", subcore_axis_name="s")

    def body(m_hbm, src_hbm, dst_hbm, w_hbm, out_hbm,
             src_b, dst_b, w_b, srcc, ldstc, wc, srcb, rows, rowsf, eidx,
             zbuf, accum, sem):
        c = lax.axis_index("c")
        s = lax.axis_index("s")
        wid = s * 2 + c
        base = wid * EW

        def zb(i, carry):
            zbuf[pl.ds(i * 16, 16)] = jnp.zeros((16,), jnp.float32)
            return carry
        lax.fori_loop(0, 128, zb, 0)

        def chunk_body(chunk, carry0):
            for z in range(CH * D // 16 // 2048):
                pltpu.sync_copy(
                    zbuf, accum.at[pl.ds(s * (CH * D // 16) + z * 2048,
                                         2048)])
            plsc.subcore_barrier()

            def block_body(eb, carry1):
                off = base + eb * EB
                pltpu.sync_copy(src_hbm.at[pl.ds(off, EB)], src_b)
                pltpu.sync_copy(dst_hbm.at[pl.ds(off, EB)], dst_b)
                pltpu.sync_copy(w_hbm.at[pl.ds(off, EB)], w_b)

                def cbody(i, n):
                    d16 = dst_b[pl.ds(i * 16, 16)]
                    if p0:
                        nid = lax.shift_right_logical(d16, 2)
                        match = ((d16 & 3) == 0) & (
                            lax.shift_right_logical(nid, CHS) == chunk)
                    else:
                        nid = d16
                        match = lax.shift_right_logical(nid, CHS) == chunk
                    s16 = src_b[pl.ds(i * 16, 16)]
                    w16 = w_b[pl.ds(i * 16, 16)]
                    l16 = nid & (CH - 1)
                    mi = jnp.where(match, 1, 0)
                    pos = n + plsc.cumsum(mi) - 1
                    plsc.store_scatter(srcc, [pos], s16, mask=match)
                    plsc.store_scatter(ldstc, [pos], l16, mask=match)
                    plsc.store_scatter(wc, [pos], w16, mask=match)
                    return n + jnp.sum(mi)
                n = lax.fori_loop(0, EB // 16, cbody, jnp.int32(0))

                for t in range(GB // 16):
                    srcc[pl.ds(n + t * 16, 16)] = jnp.zeros((16,), jnp.int32)
                    ldstc[pl.ds(n + t * 16, 16)] = jnp.zeros((16,), jnp.int32)
                    wc[pl.ds(n + t * 16, 16)] = jnp.zeros((16,), jnp.float32)
                nb = (n + GB - 1) // GB

                def gbody(bi, carry2):
                    for t in range(GB // 16):
                        srcb[pl.ds(t * 16, 16)] = (
                            srcc[pl.ds(bi * GB + t * 16, 16)])
                    pltpu.async_copy(
                        m_hbm.at[plsc.Indices(srcb)], rows, sem).wait()
                    iota = lax.iota(jnp.int32, 16)
                    for r in range(GB):
                        sel = jnp.full((16,), bi * GB + r, jnp.int32)
                        wr = plsc.load_gather(wc, [sel])
                        eb = plsc.load_gather(ldstc, [sel]) * D
                        for j in range(16):
                            rowsf[pl.ds(r * D + j * 16, 16)] = (
                                rows[r, pl.ds(j * 16, 16)] * wr)
                            eidx[pl.ds(r * D + j * 16, 16)] = (
                                eb + (iota + j * 16))
                    pltpu.sync_copy(
                        rowsf, accum.at[plsc.Indices(eidx)], add=True)
                    return carry2
                lax.fori_loop(0, nb, gbody, 0)
                return carry1
            lax.fori_loop(0, EW // EB, block_body, 0)
            plsc.subcore_barrier()

            pltpu.sync_copy(
                accum.at[pl.ds(s * (CH * D // 16), CH * D // 16)],
                out_hbm.at[c, pl.ds(chunk * CH * D + s * (CH * D // 16),
                                    CH * D // 16)])
            return carry0
        lax.fori_loop(0, nchunks, chunk_body, 0)
        plsc.subcore_barrier()

    fn = pl.kernel(
        body,
        out_type=jax.ShapeDtypeStruct((2, nout * D), jnp.float32),
        mesh=mesh,
        compiler_params=pltpu.CompilerParams(needs_layout_passes=False),
        scratch_types=[
            pltpu.VMEM((EB,), jnp.int32),
            pltpu.VMEM((EB,), jnp.int32),
            pltpu.VMEM((EB,), jnp.float32),
            pltpu.VMEM((EB + GB,), jnp.int32),
            pltpu.VMEM((EB + GB,), jnp.int32),
            pltpu.VMEM((EB + GB,), jnp.float32),
            pltpu.VMEM((GB,), jnp.int32),
            pltpu.VMEM((GB, D), jnp.float32),
            pltpu.VMEM((GB * D,), jnp.float32),
            pltpu.VMEM((GB * D,), jnp.int32),
            pltpu.VMEM((2048,), jnp.float32),
            pltpu.VMEM_SHARED((CH * D,), jnp.float32),
            pltpu.SemaphoreType.DMA,
        ],
    )
    return fn(m, src, dst, w).reshape(2, nout, D)


def _seg_jnp(m, src, dst, w, nout, p0):
    vals = w[:, None] * m[src]
    if p0:
        dstn = jnp.where((dst & 3) == 0, dst >> 2, nout)
        s = jax.ops.segment_sum(vals, dstn, num_segments=nout + 1)[:nout]
    else:
        s = jax.ops.segment_sum(vals, dst, num_segments=nout)
    return jnp.stack([s, jnp.zeros_like(s)], axis=0)


def kernel(x_user, edge_weight, params, edge_index, lengths):
    p = params
    src = edge_index[0]
    dst = edge_index[1]
    seg = _seg_jnp if _DEV_SEG_JNP else _seg_sc

    w1s1 = jnp.concatenate([p['W1'], p['Ws1']], axis=1)
    ms1 = _matmul(x_user, w1s1)
    m1 = ms1[:, :D]
    s1 = ms1[:, D:]

    part1 = seg(m1, src, dst, edge_weight, N, False)

    w2s2 = jnp.concatenate([p['W2'], p['Ws2']], axis=1)
    hw = _h_matmul(part1[0], part1[1], s1, p['b1'].reshape(1, D), w2s2)
    m2 = hw[:, :D]
    s2 = hw[:, D:]

    part2 = seg(m2, src, dst, edge_weight, BT, True)

    s2p0 = s2.reshape(BT, P, D)[:, 0, :].reshape(B, T, D)
    f0 = part2[0].reshape(B, T, D)
    f1 = part2[1].reshape(B, T, D)
    pos = p['pos_emb'][:T]

    enc_out = _encoder(lengths, f0, f1, s2p0,
                       p['b2'].reshape(1, D), p['pad_token'].reshape(1, D),
                       pos, p)
    user_node = enc_out[:, 0, :]
    user_graph = enc_out[:, 1, :]
    logits = enc_out.reshape(2 * B, D)
    sup = _head(logits, p).reshape(B, 2, PROJ)
    return user_node, user_graph, sup


# speedup vs baseline: 1.7272x; 1.1787x over previous
import functools
import math

import jax
import jax.numpy as jnp
from jax import lax
from jax.experimental import pallas as pl
from jax.experimental.pallas import tpu as pltpu
from jax.experimental.pallas import tpu_sc as plsc

B = 16; T = 512; P = 4; N = B * T * P; E = 262144
D = 256; H = 8; DH = D // H; FF = 1024; PROJ = 32
BT = B * T

_DEV_SEG_JNP = False
_INTERPRET = False


def _mm_body(x_ref, w_ref, o_ref):
    o_ref[...] = jnp.dot(x_ref[...], w_ref[...],
                         preferred_element_type=jnp.float32)


def _matmul(x, w, block_rows=1024):
    n, k = x.shape
    _, m = w.shape
    return pl.pallas_call(
        _mm_body,
        grid=(n // block_rows,),
        in_specs=[pl.BlockSpec((block_rows, k), lambda i: (i, 0)),
                  pl.BlockSpec((k, m), lambda i: (0, 0))],
        out_specs=pl.BlockSpec((block_rows, m), lambda i: (i, 0)),
        out_shape=jax.ShapeDtypeStruct((n, m), jnp.float32),
        interpret=_INTERPRET,
    )(x, w)


def _h_mm_body(p0_ref, p1_ref, s1_ref, b1_ref, w_ref, o_ref):
    h = jnp.maximum(p0_ref[...] + p1_ref[...] + s1_ref[...] + b1_ref[...], 0.0)
    o_ref[...] = jnp.dot(h, w_ref[...], preferred_element_type=jnp.float32)


def _h_matmul(p0, p1, s1, b1, w, block_rows=1024):
    n, k = p0.shape
    _, m = w.shape
    bs = lambda: pl.BlockSpec((block_rows, k), lambda i: (i, 0))
    return pl.pallas_call(
        _h_mm_body,
        grid=(n // block_rows,),
        in_specs=[bs(), bs(), bs(),
                  pl.BlockSpec((1, k), lambda i: (0, 0)),
                  pl.BlockSpec((k, m), lambda i: (0, 0))],
        out_specs=pl.BlockSpec((block_rows, m), lambda i: (i, 0)),
        out_shape=jax.ShapeDtypeStruct((n, m), jnp.float32),
        interpret=_INTERPRET,
    )(p0, p1, s1, b1, w)


def _lnk(x, g, b, eps=1e-12):
    m = jnp.mean(x, axis=-1, keepdims=True)
    v = jnp.mean((x - m) ** 2, axis=-1, keepdims=True)
    return (x - m) / jnp.sqrt(v + eps) * g + b


def _dot(a, b_):
    return jnp.dot(a, b_, preferred_element_type=jnp.float32)


def _enc_body(len_ref, f0, f1, s2, b2, padv, pos, lng, lnb,
              wq, bq, wk, bk, wv, bv, wo, bo, l1g, l1b,
              wf1, bf1, wf2, bf2, l2g, l2b, out):
    bidx = pl.program_id(0)
    L = len_ref[bidx]
    f = f0[0] + f1[0] + s2[0] + b2[...]
    tidx = lax.broadcasted_iota(jnp.int32, (T, 1), 0)
    mask = tidx < L
    x = jnp.where(mask, f, padv[...])
    x = _lnk(x + pos[...], lng[...], lnb[...])
    xln = _lnk(x, lng[...], lnb[...])
    scores = jnp.where(mask, 0.0, -10000.0).reshape(1, T)
    mf = mask.astype(jnp.float32)
    sm = jnp.maximum(jnp.sum(mf), 1e-12)
    inv = 1.0 / math.sqrt(DH)

    for a, xa in ((0, x), (1, xln)):
        q = _dot(xa, wq[...]) + bq[...]
        k = _dot(xa, wk[...]) + bk[...]
        v = _dot(xa, wv[...]) + bv[...]
        heads = []
        for h in range(H):
            qh = q[:, h * DH:(h + 1) * DH]
            kh = k[:, h * DH:(h + 1) * DH]
            vh = v[:, h * DH:(h + 1) * DH]
            att = lax.dot_general(qh, kh, (((1,), (1,)), ((), ())),
                                  preferred_element_type=jnp.float32) * inv
            att = att + scores
            att = att - jnp.max(att, axis=-1, keepdims=True)
            ew = jnp.exp(att)
            ew = ew / jnp.sum(ew, axis=-1, keepdims=True)
            heads.append(_dot(ew, vh))
        o = jnp.concatenate(heads, axis=1)
        attn = _dot(o, wo[...]) + bo[...]
        x1 = _lnk(xa + attn, l1g[...], l1b[...])
        ffh = jax.nn.gelu(_dot(x1, wf1[...]) + bf1[...])
        ff = _dot(ffh, wf2[...]) + bf2[...]
        x2 = _lnk(x1 + ff, l2g[...], l2b[...])
        out[0, a, :] = jnp.sum(x2 * mf, axis=0) / sm


def _encoder(lengths, f0, f1, s2p0, b2, padv, pos, p):
    row = lambda: pl.BlockSpec((1, T, D), lambda b: (b, 0, 0))
    cst = lambda shp: pl.BlockSpec(shp, lambda b: tuple(0 for _ in shp))
    w_names = ['Wq', 'bq', 'Wk', 'bk', 'Wv', 'bv', 'Wo', 'bo',
               'ln1_g', 'ln1_b', 'Wf1', 'bf1', 'Wf2', 'bf2', 'ln2_g', 'ln2_b']
    wvals = []
    wspecs = []
    for nm in w_names:
        arr = p[nm]
        if arr.ndim == 1:
            arr = arr.reshape(1, -1)
        wvals.append(arr)
        wspecs.append(cst(arr.shape))
    lng = p['ln_g'].reshape(1, D)
    lnb = p['ln_b'].reshape(1, D)
    return pl.pallas_call(
        _enc_body,
        grid=(B,),
        in_specs=[pl.BlockSpec(memory_space=pltpu.SMEM),
                  row(), row(), row(),
                  cst((1, D)), cst((1, D)), cst((T, D)),
                  cst((1, D)), cst((1, D))] + wspecs,
        out_specs=pl.BlockSpec((1, 2, D), lambda b: (b, 0, 0)),
        out_shape=jax.ShapeDtypeStruct((B, 2, D), jnp.float32),
        interpret=_INTERPRET,
    )(lengths, f0, f1, s2p0, b2, padv, pos, lng, lnb, *wvals)


def _head_body(lg_ref, wp1, bp1, wp2, bp2, out_ref):
    l = lg_ref[...]
    r1 = jnp.maximum(_dot(l, wp1[...]) + bp1[...], 0.0)
    pr = _dot(r1, wp2[...]) + bp2[...]
    nrm = jnp.sqrt(jnp.sum(pr * pr, axis=-1, keepdims=True))
    out_ref[...] = pr / jnp.maximum(nrm, 1e-12)


def _head(logits, p):
    return pl.pallas_call(
        _head_body,
        in_specs=[pl.BlockSpec((2 * B, D), lambda: (0, 0)),
                  pl.BlockSpec((D, D), lambda: (0, 0)),
                  pl.BlockSpec((1, D), lambda: (0, 0)),
                  pl.BlockSpec((D, PROJ), lambda: (0, 0)),
                  pl.BlockSpec((1, PROJ), lambda: (0, 0))],
        out_specs=pl.BlockSpec((2 * B, PROJ), lambda: (0, 0)),
        out_shape=jax.ShapeDtypeStruct((2 * B, PROJ), jnp.float32),
        interpret=_INTERPRET,
    )(logits, p['Wp1'].reshape(D, D), p['bp1'].reshape(1, D),
      p['Wp2'].reshape(D, PROJ), p['bp2'].reshape(1, PROJ))


EW = E // 32
CH = 2048
CHS = 11
EB = 2048
GB = 16


def _seg_sc(m, src, dst, w, nout, p0):
    nchunks = nout // CH
    mesh = plsc.VectorSubcoreMesh(core_axis_name="c", subcore_axis_name="s")

    def body(m_hbm, src_hbm, dst_hbm, w_hbm, out_hbm,
             src_b, dst_b, w_b, srcc, ldstc, wc,
             srcb0, srcb1, rows0, rows1, rowsf0, rowsf1, eidx0, eidx1,
             zbuf, accum, seme, semz, semg0, semg1, sems0, sems1):
        c = lax.axis_index("c")
        s = lax.axis_index("s")
        wid = s * 2 + c
        base = wid * EW
        srcb = (srcb0, srcb1)
        rows = (rows0, rows1)
        rowsf = (rowsf0, rowsf1)
        eidx = (eidx0, eidx1)
        semg = (semg0, semg1)
        sems = (sems0, sems1)

        def zb(i, carry):
            zbuf[pl.ds(i * 16, 16)] = jnp.zeros((16,), jnp.float32)
            return carry
        lax.fori_loop(0, 128, zb, 0)

        def chunk_body(chunk, carry0):
            zh = []
            for z in range(CH * D // 16 // 2048):
                zh.append(pltpu.async_copy(
                    zbuf, accum.at[pl.ds(s * (CH * D // 16) + z * 2048,
                                         2048)], semz))
            for h in zh:
                h.wait()
            plsc.subcore_barrier()

            def block_body(eb, carry1):
                off = base + eb * EB
                h1 = pltpu.async_copy(src_hbm.at[pl.ds(off, EB)], src_b,
                                      seme)
                h2 = pltpu.async_copy(dst_hbm.at[pl.ds(off, EB)], dst_b,
                                      seme)
                h3 = pltpu.async_copy(w_hbm.at[pl.ds(off, EB)], w_b, seme)
                h1.wait(); h2.wait(); h3.wait()

                def cbody(i, nv):
                    d16 = dst_b[pl.ds(i * 16, 16)]
                    if p0:
                        nid = lax.shift_right_logical(d16, 2)
                        match = ((d16 & 3) == 0) & (
                            lax.shift_right_logical(nid, CHS) == chunk)
                    else:
                        nid = d16
                        match = lax.shift_right_logical(nid, CHS) == chunk
                    s16 = src_b[pl.ds(i * 16, 16)]
                    w16 = w_b[pl.ds(i * 16, 16)]
                    l16 = nid & (CH - 1)
                    mi = jnp.where(match, 1, 0)
                    pos = nv + plsc.cumsum(mi) - 1
                    plsc.store_scatter(srcc, [pos], s16, mask=match)
                    plsc.store_scatter(ldstc, [pos], l16, mask=match)
                    plsc.store_scatter(wc, [pos], w16, mask=match)
                    return nv + jnp.sum(mi)
                n = lax.fori_loop(0, EB // 16, cbody, jnp.int32(0))

                for t in range(2):
                    srcc[pl.ds(n + t * 16, 16)] = jnp.zeros((16,), jnp.int32)
                    ldstc[pl.ds(n + t * 16, 16)] = jnp.zeros((16,),
                                                             jnp.int32)
                    wc[pl.ds(n + t * 16, 16)] = jnp.zeros((16,), jnp.float32)
                nbp = ((n + 2 * GB - 1) // (2 * GB)) * 2

                def stage_and_fire(k, bi):
                    srcb[k][...] = srcc[pl.ds(bi * GB, 16)]
                    pltpu.async_copy(
                        m_hbm.at[plsc.Indices(srcb[k])], rows[k], semg[k])

                for k in range(2):
                    @pl.when(nbp > k)
                    def _(k=k):
                        stage_and_fire(k, jnp.int32(k))

                iota = lax.iota(jnp.int32, 16)

                def gbody(i, carry2):
                    for k in range(2):
                        bi = 2 * i + k
                        pltpu.make_async_copy(
                            m_hbm.at[plsc.Indices(srcb[k])], rows[k],
                            semg[k]).wait()

                        @pl.when(i > 0)
                        def _(k=k):
                            pltpu.make_async_copy(
                                rowsf[k],
                                accum.at[plsc.Indices(eidx[k])],
                                sems[k]).wait()
                        for r in range(GB):
                            sel = bi * GB + r
                            selv = jnp.full((16,), sel, jnp.int32)
                            wr = plsc.load_gather(wc, [selv])
                            ebase = plsc.load_gather(ldstc, [selv]) * D
                            for j in range(16):
                                rowsf[k][pl.ds(r * D + j * 16, 16)] = (
                                    rows[k][r, pl.ds(j * 16, 16)] * wr)
                                eidx[k][pl.ds(r * D + j * 16, 16)] = (
                                    ebase + (iota + j * 16))
                        pltpu.async_copy(
                            rowsf[k], accum.at[plsc.Indices(eidx[k])],
                            sems[k])

                        @pl.when(bi + 2 < nbp)
                        def _(k=k, bi=bi):
                            stage_and_fire(k, bi + 2)
                    return carry2
                lax.fori_loop(0, nbp // 2, gbody, 0)
                for k in range(2):
                    @pl.when(nbp > 0)
                    def _(k=k):
                        pltpu.make_async_copy(
                            rowsf[k], accum.at[plsc.Indices(eidx[k])],
                            sems[k]).wait()
                return carry1
            lax.fori_loop(0, EW // EB, block_body, 0)
            plsc.subcore_barrier()

            pltpu.sync_copy(
                accum.at[pl.ds(s * (CH * D // 16), CH * D // 16)],
                out_hbm.at[c, pl.ds(chunk * CH * D + s * (CH * D // 16),
                                    CH * D // 16)])
            return carry0
        lax.fori_loop(0, nchunks, chunk_body, 0)
        plsc.subcore_barrier()

    fn = pl.kernel(
        body,
        out_type=jax.ShapeDtypeStruct((2, nout * D), jnp.float32),
        mesh=mesh,
        compiler_params=pltpu.CompilerParams(needs_layout_passes=False),
        scratch_types=[
            pltpu.VMEM((EB,), jnp.int32),
            pltpu.VMEM((EB,), jnp.int32),
            pltpu.VMEM((EB,), jnp.float32),
            pltpu.VMEM((EB + 32,), jnp.int32),
            pltpu.VMEM((EB + 32,), jnp.int32),
            pltpu.VMEM((EB + 32,), jnp.float32),
            pltpu.VMEM((GB,), jnp.int32),
            pltpu.VMEM((GB,), jnp.int32),
            pltpu.VMEM((GB, D), jnp.float32),
            pltpu.VMEM((GB, D), jnp.float32),
            pltpu.VMEM((GB * D,), jnp.float32),
            pltpu.VMEM((GB * D,), jnp.float32),
            pltpu.VMEM((GB * D,), jnp.int32),
            pltpu.VMEM((GB * D,), jnp.int32),
            pltpu.VMEM((2048,), jnp.float32),
            pltpu.VMEM_SHARED((CH * D,), jnp.float32),
            pltpu.SemaphoreType.DMA,
            pltpu.SemaphoreType.DMA,
            pltpu.SemaphoreType.DMA,
            pltpu.SemaphoreType.DMA,
            pltpu.SemaphoreType.DMA,
            pltpu.SemaphoreType.DMA,
        ],
    )
    return fn(m, src, dst, w).reshape(2, nout, D)


def _seg_jnp(m, src, dst, w, nout, p0):
    vals = w[:, None] * m[src]
    if p0:
        dstn = jnp.where((dst & 3) == 0, dst >> 2, nout)
        s = jax.ops.segment_sum(vals, dstn, num_segments=nout + 1)[:nout]
    else:
        s = jax.ops.segment_sum(vals, dst, num_segments=nout)
    return jnp.stack([s, jnp.zeros_like(s)], axis=0)


def kernel(x_user, edge_weight, params, edge_index, lengths):
    p = params
    src = edge_index[0]
    dst = edge_index[1]
    seg = _seg_jnp if _DEV_SEG_JNP else _seg_sc

    w1s1 = jnp.concatenate([p['W1'], p['Ws1']], axis=1)
    ms1 = _matmul(x_user, w1s1)
    m1 = ms1[:, :D]
    s1 = ms1[:, D:]

    part1 = seg(m1, src, dst, edge_weight, N, False)

    w2s2 = jnp.concatenate([p['W2'], p['Ws2']], axis=1)
    hw = _h_matmul(part1[0], part1[1], s1, p['b1'].reshape(1, D), w2s2)
    m2 = hw[:, :D]
    s2 = hw[:, D:]

    part2 = seg(m2, src, dst, edge_weight, BT, True)

    s2p0 = s2.reshape(BT, P, D)[:, 0, :].reshape(B, T, D)
    f0 = part2[0].reshape(B, T, D)
    f1 = part2[1].reshape(B, T, D)
    pos = p['pos_emb'][:T]

    enc_out = _encoder(lengths, f0, f1, s2p0,
                       p['b2'].reshape(1, D), p['pad_token'].reshape(1, D),
                       pos, p)
    user_node = enc_out[:, 0, :]
    user_graph = enc_out[:, 1, :]
    logits = enc_out.reshape(2 * B, D)
    sup = _head(logits, p).reshape(B, 2, PROJ)
    return user_node, user_graph, sup
